# baseline (speedup 1.0000x reference)
import functools

import jax
import jax.numpy as jnp
from jax import lax
from jax.experimental import pallas as pl
from jax.experimental.pallas import tpu as pltpu
from jax.experimental.pallas import tpu_sc as plsc

NC = 2
NS = 16
L = 16
NW = NC * NS
D = 256
G = 16
DC = D // L
N = 50000
CHUNK = 1560
NBLK = 13
BR = CHUNK // NBLK
TAIL = N - NW * CHUNK

_mesh = plsc.VectorSubcoreMesh(core_axis_name="c", subcore_axis_name="s")


@functools.partial(
    pl.kernel,
    out_type=jax.ShapeDtypeStruct((NC, G, D), jnp.float32),
    mesh=_mesh,
    scratch_types=[
        pltpu.VMEM((CHUNK + L,), jnp.int32),
        pltpu.VMEM((TAIL + L,), jnp.int32),
        pltpu.VMEM((2, BR, D), jnp.float32),
        pltpu.VMEM((G, D), jnp.float32),
        pltpu.VMEM((G, D), jnp.float32),
        pltpu.VMEM_SHARED((NS, G, D), jnp.float32),
        pltpu.SemaphoreType.DMA,
        pltpu.SemaphoreType.DMA,
    ],
)
def _segment_pool(nodes_h, idx_h, out_h, idx_v, idx_t, buf_v, acc_v, tmp_v,
                  shared, sem0, sem1):
    cid = lax.axis_index("c")
    sid = lax.axis_index("s")
    wid = cid * NS + sid
    base = wid * CHUNK

    zeros = jnp.zeros((L,), jnp.float32)

    def _zero(i, _):
        for c in range(DC):
            acc_v[i, pl.ds(c * L, L)] = zeros
        return 0

    lax.fori_loop(0, G, _zero, 0)

    pltpu.sync_copy(idx_h.at[pl.ds(base, CHUNK)], idx_v.at[pl.ds(0, CHUNK)])

    def _accum_rows(idx_ref, idx_off, n_rows, b):
        def _row(i, _):
            g = idx_ref[pl.ds(idx_off + i, L)][0]
            for c in range(DC):
                plsc.addupdate(acc_v.at[g, pl.ds(c * L, L)],
                               buf_v[b, i, pl.ds(c * L, L)])
            return 0

        lax.fori_loop(0, n_rows, _row, 0, unroll=8)

    def _node_copy(blk, b):
        return pltpu.make_async_copy(
            nodes_h.at[pl.ds(base + blk * BR, BR)],
            buf_v.at[b],
            sem0 if b == 0 else sem1)

    _node_copy(0, 0).start()
    for blk in range(NBLK):
        b = blk % 2
        cp = _node_copy(blk, b)
        if blk + 1 < NBLK:
            _node_copy(blk + 1, (blk + 1) % 2).start()
        cp.wait()
        _accum_rows(idx_v, blk * BR, BR, b)

    @pl.when(wid == NW - 1)
    def _():
        pltpu.sync_copy(idx_h.at[pl.ds(NW * CHUNK, TAIL)],
                        idx_t.at[pl.ds(0, TAIL)])
        pltpu.sync_copy(nodes_h.at[pl.ds(NW * CHUNK, TAIL)],
                        buf_v.at[0, pl.ds(0, TAIL)])
        _accum_rows(idx_t, 0, TAIL, 0)

    pltpu.sync_copy(acc_v, shared.at[sid])
    plsc.subcore_barrier()

    def _acc_add(i, _):
        for c in range(DC):
            acc_v[i, pl.ds(c * L, L)] = (acc_v[i, pl.ds(c * L, L)] +
                                         tmp_v[i, pl.ds(c * L, L)])
        return 0

    for step in (8, 4, 2, 1):
        @pl.when(sid < step)
        def _(step=step):
            pltpu.sync_copy(shared.at[sid + step], tmp_v)
            lax.fori_loop(0, G, _acc_add, 0)
            pltpu.sync_copy(acc_v, shared.at[sid])

        plsc.subcore_barrier()

    @pl.when(sid == 0)
    def _():
        pltpu.sync_copy(acc_v, out_h.at[cid])


@functools.partial(
    pl.kernel,
    out_type=jax.ShapeDtypeStruct((G,), jnp.float32),
    mesh=_mesh,
    scratch_types=[
        pltpu.VMEM((NC, G, D), jnp.float32),
        pltpu.VMEM((G, D), jnp.float32),
        pltpu.VMEM((D,), jnp.float32),
        pltpu.VMEM((D,), jnp.float32),
        pltpu.VMEM((G,), jnp.float32),
        pltpu.VMEM((G,), jnp.float32),
    ],
)
def _decode(parts_h, glob_h, wp_h, wg_h, b_h, out_h, parts_v, glob_v, wp_v,
            wg_v, b_v, out_v):
    cid = lax.axis_index("c")
    sid = lax.axis_index("s")

    @pl.when((cid == 0) & (sid == 0))
    def _():
        pltpu.sync_copy(parts_h, parts_v)
        pltpu.sync_copy(glob_h, glob_v)
        pltpu.sync_copy(wp_h, wp_v)
        pltpu.sync_copy(wg_h, wg_v)
        pltpu.sync_copy(b_h, b_v)

        lane = lax.iota(jnp.int32, L)
        lv = b_v[...]
        for g in range(G):
            def _c(c, pv, g=g):
                p = (parts_v[0, g, pl.ds(c * L, L)] +
                     parts_v[1, g, pl.ds(c * L, L)])
                pv = pv + p * wp_v[pl.ds(c * L, L)]
                pv = pv + (glob_v[g, pl.ds(c * L, L)] *
                           wg_v[pl.ds(c * L, L)])
                return pv

            pv = lax.fori_loop(0, DC, _c, jnp.zeros((L,), jnp.float32))
            s = pv[0]
            for j in range(1, L):
                s = s + pv[j]
            lv = jnp.where(lane == g, lv + s, lv)
        out_v[...] = lv
        pltpu.sync_copy(out_v, out_h)


def kernel(nodes, edges, senders, receivers, global_latent, node_graph_idx,
           W, b):
    idx = node_graph_idx.astype(jnp.int32)
    parts = _segment_pool(nodes, idx)
    wp = W[:D, 0].astype(jnp.float32)
    wg = W[D:, 0].astype(jnp.float32)
    bb = jnp.broadcast_to(b.astype(jnp.float32), (G,))
    logits = _decode(parts, global_latent, wp, wg, bb)
    return logits.reshape(G, 1)


# speedup vs baseline: 2.1075x; 2.1075x over previous
import functools

import jax
import jax.numpy as jnp
from jax import lax
from jax.experimental import pallas as pl
from jax.experimental.pallas import tpu as pltpu
from jax.experimental.pallas import tpu_sc as plsc

NC = 2
NS = 16
L = 16
NW = NC * NS
D = 256
G = 16
DC = D // L
N = 50000
CHUNK = 1560
NBLK = 13
BR = CHUNK // NBLK
TAIL = N - NW * CHUNK

_mesh = plsc.VectorSubcoreMesh(core_axis_name="c", subcore_axis_name="s")


@functools.partial(
    pl.kernel,
    out_type=jax.ShapeDtypeStruct((NC, G, D), jnp.float32),
    mesh=_mesh,
    scratch_types=[
        pltpu.VMEM((CHUNK + L,), jnp.int32),
        pltpu.VMEM((TAIL + L,), jnp.int32),
        pltpu.VMEM((2, BR, D), jnp.float32),
        pltpu.VMEM((G, D), jnp.float32),
        pltpu.VMEM((G, D), jnp.float32),
        pltpu.VMEM_SHARED((NS, G, D), jnp.float32),
        pltpu.SemaphoreType.DMA,
        pltpu.SemaphoreType.DMA,
    ],
)
def _segment_pool(nodes_h, idx_h, out_h, idx_v, idx_t, buf_v, acc_v, tmp_v,
                  shared, sem0, sem1):
    cid = lax.axis_index("c")
    sid = lax.axis_index("s")
    wid = cid * NS + sid
    base = wid * CHUNK

    zeros = jnp.zeros((L,), jnp.float32)

    def _zero(i, _):
        for c in range(DC):
            acc_v[i, pl.ds(c * L, L)] = zeros
        return 0

    lax.fori_loop(0, G, _zero, 0)

    pltpu.sync_copy(idx_h.at[pl.ds(base, CHUNK)], idx_v.at[pl.ds(0, CHUNK)])

    zregs = tuple(jnp.zeros((L,), jnp.float32) for _ in range(DC))

    def _accum_rows(idx_ref, idx_off, n_rows, b):
        g_first = idx_ref[pl.ds(idx_off, L)][0]
        g_last = idx_ref[pl.ds(idx_off + n_rows - 1, L)][0]

        @pl.when(g_first == g_last)
        def _():
            def _row(i, regs):
                return tuple(regs[c] + buf_v[b, i, pl.ds(c * L, L)]
                             for c in range(DC))

            regs = lax.fori_loop(0, n_rows, _row, zregs, unroll=4)
            for c in range(DC):
                plsc.addupdate(acc_v.at[g_first, pl.ds(c * L, L)], regs[c])

        @pl.when(g_first != g_last)
        def _():
            def _row(i, _):
                g = idx_ref[pl.ds(idx_off + i, L)][0]
                for c in range(DC):
                    plsc.addupdate(acc_v.at[g, pl.ds(c * L, L)],
                                   buf_v[b, i, pl.ds(c * L, L)])
                return 0

            lax.fori_loop(0, n_rows, _row, 0)

    def _node_copy(blk, b):
        return pltpu.make_async_copy(
            nodes_h.at[pl.ds(base + blk * BR, BR)],
            buf_v.at[b],
            sem0 if b == 0 else sem1)

    _node_copy(0, 0).start()
    for blk in range(NBLK):
        b = blk % 2
        cp = _node_copy(blk, b)
        if blk + 1 < NBLK:
            _node_copy(blk + 1, (blk + 1) % 2).start()
        cp.wait()
        _accum_rows(idx_v, blk * BR, BR, b)

    @pl.when(wid == NW - 1)
    def _():
        pltpu.sync_copy(idx_h.at[pl.ds(NW * CHUNK, TAIL)],
                        idx_t.at[pl.ds(0, TAIL)])
        pltpu.sync_copy(nodes_h.at[pl.ds(NW * CHUNK, TAIL)],
                        buf_v.at[0, pl.ds(0, TAIL)])
        _accum_rows(idx_t, 0, TAIL, 0)

    pltpu.sync_copy(acc_v, shared.at[sid])
    plsc.subcore_barrier()

    def _acc_add(i, _):
        for c in range(DC):
            acc_v[i, pl.ds(c * L, L)] = (acc_v[i, pl.ds(c * L, L)] +
                                         tmp_v[i, pl.ds(c * L, L)])
        return 0

    for step in (8, 4, 2, 1):
        @pl.when(sid < step)
        def _(step=step):
            pltpu.sync_copy(shared.at[sid + step], tmp_v)
            lax.fori_loop(0, G, _acc_add, 0)
            pltpu.sync_copy(acc_v, shared.at[sid])

        plsc.subcore_barrier()

    @pl.when(sid == 0)
    def _():
        pltpu.sync_copy(acc_v, out_h.at[cid])


@functools.partial(
    pl.kernel,
    out_type=jax.ShapeDtypeStruct((G,), jnp.float32),
    mesh=_mesh,
    scratch_types=[
        pltpu.VMEM((NC, G, D), jnp.float32),
        pltpu.VMEM((G, D), jnp.float32),
        pltpu.VMEM((D,), jnp.float32),
        pltpu.VMEM((D,), jnp.float32),
        pltpu.VMEM((G,), jnp.float32),
        pltpu.VMEM((G,), jnp.float32),
    ],
)
def _decode(parts_h, glob_h, wp_h, wg_h, b_h, out_h, parts_v, glob_v, wp_v,
            wg_v, b_v, out_v):
    cid = lax.axis_index("c")
    sid = lax.axis_index("s")

    @pl.when((cid == 0) & (sid == 0))
    def _():
        pltpu.sync_copy(parts_h, parts_v)
        pltpu.sync_copy(glob_h, glob_v)
        pltpu.sync_copy(wp_h, wp_v)
        pltpu.sync_copy(wg_h, wg_v)
        pltpu.sync_copy(b_h, b_v)

        lane = lax.iota(jnp.int32, L)
        lv = b_v[...]
        for g in range(G):
            def _c(c, pv, g=g):
                p = (parts_v[0, g, pl.ds(c * L, L)] +
                     parts_v[1, g, pl.ds(c * L, L)])
                pv = pv + p * wp_v[pl.ds(c * L, L)]
                pv = pv + (glob_v[g, pl.ds(c * L, L)] *
                           wg_v[pl.ds(c * L, L)])
                return pv

            pv = lax.fori_loop(0, DC, _c, jnp.zeros((L,), jnp.float32))
            s = pv[0]
            for j in range(1, L):
                s = s + pv[j]
            lv = jnp.where(lane == g, lv + s, lv)
        out_v[...] = lv
        pltpu.sync_copy(out_v, out_h)


def kernel(nodes, edges, senders, receivers, global_latent, node_graph_idx,
           W, b):
    idx = node_graph_idx.astype(jnp.int32)
    parts = _segment_pool(nodes, idx)
    wp = W[:D, 0].astype(jnp.float32)
    wg = W[D:, 0].astype(jnp.float32)
    bb = jnp.broadcast_to(b.astype(jnp.float32), (G,))
    logits = _decode(parts, global_latent, wp, wg, bb)
    return logits.reshape(G, 1)


# speedup vs baseline: 2.2660x; 1.0752x over previous
import functools

import jax
import jax.numpy as jnp
from jax import lax
from jax.experimental import pallas as pl
from jax.experimental.pallas import tpu as pltpu
from jax.experimental.pallas import tpu_sc as plsc

NC = 2
NS = 16
L = 16
NW = NC * NS
D = 256
G = 16
DC = D // L
N = 50000
CHUNK = 1560
NBLK = 13
BR = CHUNK // NBLK
TAIL = N - NW * CHUNK

_mesh = plsc.VectorSubcoreMesh(core_axis_name="c", subcore_axis_name="s")


@functools.partial(
    pl.kernel,
    out_type=jax.ShapeDtypeStruct((NC, G), jnp.float32),
    mesh=_mesh,
    scratch_types=[
        pltpu.VMEM((CHUNK + L,), jnp.int32),
        pltpu.VMEM((TAIL + L,), jnp.int32),
        pltpu.VMEM((2, BR, D), jnp.float32),
        pltpu.VMEM((G, D), jnp.float32),
        pltpu.VMEM((G, D), jnp.float32),
        pltpu.VMEM((G, D), jnp.float32),
        pltpu.VMEM((D,), jnp.float32),
        pltpu.VMEM((D,), jnp.float32),
        pltpu.VMEM((G,), jnp.float32),
        pltpu.VMEM((G,), jnp.float32),
        pltpu.VMEM_SHARED((NS, G, D), jnp.float32),
        pltpu.SemaphoreType.DMA,
        pltpu.SemaphoreType.DMA,
    ],
)
def _pool_decode(nodes_h, idx_h, glob_h, wp_h, wg_h, b_h, out_h,
                 idx_v, idx_t, buf_v, acc_v, tmp_v, glob_v, wp_v, wg_v, b_v,
                 out_v, shared, sem0, sem1):
    cid = lax.axis_index("c")
    sid = lax.axis_index("s")
    wid = cid * NS + sid
    base = wid * CHUNK

    def _node_copy(blk, b):
        return pltpu.make_async_copy(
            nodes_h.at[pl.ds(base + blk * BR, BR)],
            buf_v.at[b],
            sem0 if b == 0 else sem1)

    _node_copy(0, 0).start()
    pltpu.sync_copy(idx_h.at[pl.ds(base, CHUNK)], idx_v.at[pl.ds(0, CHUNK)])

    zeros = jnp.zeros((L,), jnp.float32)

    def _zero(i, _):
        for c in range(DC):
            acc_v[i, pl.ds(c * L, L)] = zeros
        return 0

    lax.fori_loop(0, G, _zero, 0)

    zregs = tuple(jnp.zeros((L,), jnp.float32) for _ in range(DC))

    def _accum_rows(idx_ref, idx_off, n_rows, b):
        g_first = idx_ref[pl.ds(idx_off, L)][0]
        g_last = idx_ref[pl.ds(idx_off + n_rows - 1, L)][0]

        @pl.when(g_first == g_last)
        def _():
            def _row(i, regs):
                return tuple(regs[c] + buf_v[b, i, pl.ds(c * L, L)]
                             for c in range(DC))

            regs = lax.fori_loop(0, n_rows, _row, zregs, unroll=4)
            for c in range(DC):
                plsc.addupdate(acc_v.at[g_first, pl.ds(c * L, L)], regs[c])

        @pl.when(g_first != g_last)
        def _():
            def _row(i, _):
                g = idx_ref[pl.ds(idx_off + i, L)][0]
                for c in range(DC):
                    plsc.addupdate(acc_v.at[g, pl.ds(c * L, L)],
                                   buf_v[b, i, pl.ds(c * L, L)])
                return 0

            lax.fori_loop(0, n_rows, _row, 0)

    for blk in range(NBLK):
        b = blk % 2
        cp = _node_copy(blk, b)
        if blk + 1 < NBLK:
            _node_copy(blk + 1, (blk + 1) % 2).start()
        cp.wait()
        _accum_rows(idx_v, blk * BR, BR, b)

    @pl.when(wid == NW - 1)
    def _():
        pltpu.sync_copy(idx_h.at[pl.ds(NW * CHUNK, TAIL)],
                        idx_t.at[pl.ds(0, TAIL)])
        pltpu.sync_copy(nodes_h.at[pl.ds(NW * CHUNK, TAIL)],
                        buf_v.at[0, pl.ds(0, TAIL)])
        _accum_rows(idx_t, 0, TAIL, 0)

    pltpu.sync_copy(acc_v, shared.at[sid])
    plsc.subcore_barrier()

    def _acc_add(i, _):
        for c in range(DC):
            acc_v[i, pl.ds(c * L, L)] = (acc_v[i, pl.ds(c * L, L)] +
                                         tmp_v[i, pl.ds(c * L, L)])
        return 0

    for step in (8, 4, 2, 1):
        @pl.when(sid < step)
        def _(step=step):
            pltpu.sync_copy(shared.at[sid + step], tmp_v)
            lax.fori_loop(0, G, _acc_add, 0)
            pltpu.sync_copy(acc_v, shared.at[sid])

        plsc.subcore_barrier()

    lane = lax.iota(jnp.int32, L)

    def _decode_partial(with_glob):
        lv = b_v[...] if with_glob else zeros
        for g in range(G):
            def _c(c, pv, g=g):
                pv = pv + acc_v[g, pl.ds(c * L, L)] * wp_v[pl.ds(c * L, L)]
                if with_glob:
                    pv = pv + (glob_v[g, pl.ds(c * L, L)] *
                               wg_v[pl.ds(c * L, L)])
                return pv

            pv = lax.fori_loop(0, DC, _c, jnp.zeros((L,), jnp.float32))
            s = pv[0]
            for j in range(1, L):
                s = s + pv[j]
            lv = jnp.where(lane == g, lv + s, lv)
        return lv

    @pl.when((sid == 0) & (cid == 0))
    def _():
        pltpu.sync_copy(glob_h, glob_v)
        pltpu.sync_copy(wp_h, wp_v)
        pltpu.sync_copy(wg_h, wg_v)
        pltpu.sync_copy(b_h, b_v)
        out_v[...] = _decode_partial(True)
        pltpu.sync_copy(out_v, out_h.at[0])

    @pl.when((sid == 0) & (cid == 1))
    def _():
        pltpu.sync_copy(wp_h, wp_v)
        out_v[...] = _decode_partial(False)
        pltpu.sync_copy(out_v, out_h.at[1])


def kernel(nodes, edges, senders, receivers, global_latent, node_graph_idx,
           W, b):
    idx = node_graph_idx.astype(jnp.int32)
    wp = W[:D, 0].astype(jnp.float32)
    wg = W[D:, 0].astype(jnp.float32)
    bb = jnp.broadcast_to(b.astype(jnp.float32), (G,))
    parts = _pool_decode(nodes, idx, global_latent, wp, wg, bb)
    return (parts[0] + parts[1]).reshape(G, 1)


# speedup vs baseline: 2.3955x; 1.0571x over previous
import functools

import jax
import jax.numpy as jnp
from jax import lax
from jax.experimental import pallas as pl
from jax.experimental.pallas import tpu as pltpu
from jax.experimental.pallas import tpu_sc as plsc

NC = 2
NS = 16
L = 16
NW = NC * NS
D = 256
G = 16
DC = D // L
N = 50000
CHUNK = 1560
NBLK = 13
BR = CHUNK // NBLK
TAIL = N - NW * CHUNK

_mesh = plsc.VectorSubcoreMesh(core_axis_name="c", subcore_axis_name="s")


@functools.partial(
    pl.kernel,
    out_type=jax.ShapeDtypeStruct((NC, G), jnp.float32),
    mesh=_mesh,
    scratch_types=[
        pltpu.VMEM((CHUNK + L,), jnp.int32),
        pltpu.VMEM((TAIL + L,), jnp.int32),
        pltpu.VMEM((3, BR, D), jnp.float32),
        pltpu.VMEM((G, D), jnp.float32),
        pltpu.VMEM((G, D), jnp.float32),
        pltpu.VMEM((G, D), jnp.float32),
        pltpu.VMEM((D,), jnp.float32),
        pltpu.VMEM((D,), jnp.float32),
        pltpu.VMEM((G,), jnp.float32),
        pltpu.VMEM((G,), jnp.float32),
        pltpu.VMEM_SHARED((NS, G, D), jnp.float32),
        pltpu.SemaphoreType.DMA,
        pltpu.SemaphoreType.DMA,
        pltpu.SemaphoreType.DMA,
    ],
)
def _pool_decode(nodes_h, idx_h, glob_h, wp_h, wg_h, b_h, out_h,
                 idx_v, idx_t, buf_v, acc_v, tmp_v, glob_v, wp_v, wg_v, b_v,
                 out_v, shared, sem0, sem1, sem2):
    cid = lax.axis_index("c")
    sid = lax.axis_index("s")
    wid = cid * NS + sid
    base = wid * CHUNK
    sems = (sem0, sem1, sem2)

    def _node_copy(blk, b):
        return pltpu.make_async_copy(
            nodes_h.at[pl.ds(base + blk * BR, BR)],
            buf_v.at[b],
            sems[b])

    _node_copy(0, 0).start()
    _node_copy(1, 1).start()
    pltpu.sync_copy(idx_h.at[pl.ds(base, CHUNK)], idx_v.at[pl.ds(0, CHUNK)])

    zeros = jnp.zeros((L,), jnp.float32)

    def _zero(i, _):
        for c in range(DC):
            acc_v[i, pl.ds(c * L, L)] = zeros
        return 0

    lax.fori_loop(0, G, _zero, 0)

    zregs = tuple(jnp.zeros((L,), jnp.float32) for _ in range(DC))

    def _accum_rows(idx_ref, idx_off, n_rows, b):
        g_first = idx_ref[pl.ds(idx_off, L)][0]
        g_last = idx_ref[pl.ds(idx_off + n_rows - 1, L)][0]

        @pl.when(g_first == g_last)
        def _():
            def _row(i, regs):
                return tuple(regs[c] + buf_v[b, i, pl.ds(c * L, L)]
                             for c in range(DC))

            regs = lax.fori_loop(0, n_rows, _row, zregs, unroll=4)
            for c in range(DC):
                plsc.addupdate(acc_v.at[g_first, pl.ds(c * L, L)], regs[c])

        @pl.when(g_first != g_last)
        def _():
            def _row(i, _):
                g = idx_ref[pl.ds(idx_off + i, L)][0]
                for c in range(DC):
                    plsc.addupdate(acc_v.at[g, pl.ds(c * L, L)],
                                   buf_v[b, i, pl.ds(c * L, L)])
                return 0

            lax.fori_loop(0, n_rows, _row, 0)

    for blk in range(NBLK):
        b = blk % 3
        cp = _node_copy(blk, b)
        if blk + 2 < NBLK:
            _node_copy(blk + 2, (blk + 2) % 3).start()
        cp.wait()
        _accum_rows(idx_v, blk * BR, BR, b)

    @pl.when(wid == NW - 1)
    def _():
        pltpu.sync_copy(idx_h.at[pl.ds(NW * CHUNK, TAIL)],
                        idx_t.at[pl.ds(0, TAIL)])
        pltpu.sync_copy(nodes_h.at[pl.ds(NW * CHUNK, TAIL)],
                        buf_v.at[0, pl.ds(0, TAIL)])
        _accum_rows(idx_t, 0, TAIL, 0)

    pltpu.sync_copy(acc_v, shared.at[sid])
    plsc.subcore_barrier()

    def _acc_add(i, _):
        for c in range(DC):
            acc_v[i, pl.ds(c * L, L)] = (acc_v[i, pl.ds(c * L, L)] +
                                         tmp_v[i, pl.ds(c * L, L)])
        return 0

    for step in (8, 4, 2, 1):
        @pl.when(sid < step)
        def _(step=step):
            pltpu.sync_copy(shared.at[sid + step], tmp_v)
            lax.fori_loop(0, G, _acc_add, 0)
            pltpu.sync_copy(acc_v, shared.at[sid])

        plsc.subcore_barrier()

    lane = lax.iota(jnp.int32, L)

    def _decode_partial(with_glob):
        lv = b_v[...] if with_glob else zeros
        for g in range(G):
            def _c(c, pv, g=g):
                pv = pv + acc_v[g, pl.ds(c * L, L)] * wp_v[pl.ds(c * L, L)]
                if with_glob:
                    pv = pv + (glob_v[g, pl.ds(c * L, L)] *
                               wg_v[pl.ds(c * L, L)])
                return pv

            pv = lax.fori_loop(0, DC, _c, jnp.zeros((L,), jnp.float32))
            s = pv[0]
            for j in range(1, L):
                s = s + pv[j]
            lv = jnp.where(lane == g, lv + s, lv)
        return lv

    @pl.when((sid == 0) & (cid == 0))
    def _():
        pltpu.sync_copy(glob_h, glob_v)
        pltpu.sync_copy(wp_h, wp_v)
        pltpu.sync_copy(wg_h, wg_v)
        pltpu.sync_copy(b_h, b_v)
        out_v[...] = _decode_partial(True)
        pltpu.sync_copy(out_v, out_h.at[0])

    @pl.when((sid == 0) & (cid == 1))
    def _():
        pltpu.sync_copy(wp_h, wp_v)
        out_v[...] = _decode_partial(False)
        pltpu.sync_copy(out_v, out_h.at[1])


def kernel(nodes, edges, senders, receivers, global_latent, node_graph_idx,
           W, b):
    idx = node_graph_idx.astype(jnp.int32)
    wp = W[:D, 0].astype(jnp.float32)
    wg = W[D:, 0].astype(jnp.float32)
    bb = jnp.broadcast_to(b.astype(jnp.float32), (G,))
    parts = _pool_decode(nodes, idx, global_latent, wp, wg, bb)
    return (parts[0] + parts[1]).reshape(G, 1)


# speedup vs baseline: 3.1014x; 1.2947x over previous
import functools

import jax
import jax.numpy as jnp
from jax import lax
from jax.experimental import pallas as pl
from jax.experimental.pallas import tpu as pltpu
from jax.experimental.pallas import tpu_sc as plsc

NC = 2
NS = 16
L = 16
NW = NC * NS
D = 256
G = 16
DC = D // L
N = 50000

BT = 2048
NBT = 17
NTC = NBT * BT
CHUNK = 472
SZS = (120, 120, 120, 112)
BR = 120
TAIL = N - NTC - NW * CHUNK

_mesh = plsc.VectorSubcoreMesh(core_axis_name="c", subcore_axis_name="s")


@functools.partial(
    pl.kernel,
    out_type=jax.ShapeDtypeStruct((NC, G), jnp.float32),
    mesh=_mesh,
    scratch_types=[
        pltpu.VMEM((CHUNK + L,), jnp.int32),
        pltpu.VMEM((TAIL + L,), jnp.int32),
        pltpu.VMEM((3, BR, D), jnp.float32),
        pltpu.VMEM((G, D), jnp.float32),
        pltpu.VMEM((G, D), jnp.float32),
        pltpu.VMEM((D,), jnp.float32),
        pltpu.VMEM((G,), jnp.float32),
        pltpu.VMEM_SHARED((NS, G, D), jnp.float32),
        pltpu.SemaphoreType.DMA,
        pltpu.SemaphoreType.DMA,
        pltpu.SemaphoreType.DMA,
    ],
)
def _sc_pool(nodes_h, idx_h, wp_h, out_h,
             idx_v, idx_t, buf_v, acc_v, tmp_v, wp_v, out_v,
             shared, sem0, sem1, sem2):
    cid = lax.axis_index("c")
    sid = lax.axis_index("s")
    wid = cid * NS + sid
    base = NTC + wid * CHUNK
    sems = (sem0, sem1, sem2)
    offs = [sum(SZS[:k]) for k in range(len(SZS))]

    def _node_copy(blk, b):
        return pltpu.make_async_copy(
            nodes_h.at[pl.ds(base + offs[blk], SZS[blk])],
            buf_v.at[b, pl.ds(0, SZS[blk])],
            sems[b])

    _node_copy(0, 0).start()
    _node_copy(1, 1).start()
    pltpu.sync_copy(idx_h.at[pl.ds(base, CHUNK)], idx_v.at[pl.ds(0, CHUNK)])

    zeros = jnp.zeros((L,), jnp.float32)

    def _zero(i, _):
        for c in range(DC):
            acc_v[i, pl.ds(c * L, L)] = zeros
        return 0

    lax.fori_loop(0, G, _zero, 0)

    zregs = tuple(jnp.zeros((L,), jnp.float32) for _ in range(DC))

    def _accum_rows(idx_ref, idx_off, n_rows, b):
        g_first = idx_ref[pl.ds(idx_off, L)][0]
        g_last = idx_ref[pl.ds(idx_off + n_rows - 1, L)][0]

        @pl.when(g_first == g_last)
        def _():
            def _row(i, regs):
                return tuple(regs[c] + buf_v[b, i, pl.ds(c * L, L)]
                             for c in range(DC))

            regs = lax.fori_loop(0, n_rows, _row, zregs, unroll=4)
            for c in range(DC):
                plsc.addupdate(acc_v.at[g_first, pl.ds(c * L, L)], regs[c])

        @pl.when(g_first != g_last)
        def _():
            def _row(i, _):
                g = idx_ref[pl.ds(idx_off + i, L)][0]
                for c in range(DC):
                    plsc.addupdate(acc_v.at[g, pl.ds(c * L, L)],
                                   buf_v[b, i, pl.ds(c * L, L)])
                return 0

            lax.fori_loop(0, n_rows, _row, 0)

    for blk in range(len(SZS)):
        b = blk % 3
        cp = _node_copy(blk, b)
        if blk + 2 < len(SZS):
            _node_copy(blk + 2, (blk + 2) % 3).start()
        cp.wait()
        _accum_rows(idx_v, offs[blk], SZS[blk], b)

    @pl.when(wid == NW - 1)
    def _():
        pltpu.sync_copy(idx_h.at[pl.ds(N - TAIL, TAIL)],
                        idx_t.at[pl.ds(0, TAIL)])
        pltpu.sync_copy(nodes_h.at[pl.ds(N - TAIL, TAIL)],
                        buf_v.at[0, pl.ds(0, TAIL)])
        _accum_rows(idx_t, 0, TAIL, 0)

    pltpu.sync_copy(acc_v, shared.at[sid])
    plsc.subcore_barrier()

    def _acc_add(i, _):
        for c in range(DC):
            acc_v[i, pl.ds(c * L, L)] = (acc_v[i, pl.ds(c * L, L)] +
                                         tmp_v[i, pl.ds(c * L, L)])
        return 0

    for step in (8, 4, 2, 1):
        @pl.when(sid < step)
        def _(step=step):
            pltpu.sync_copy(shared.at[sid + step], tmp_v)
            lax.fori_loop(0, G, _acc_add, 0)
            pltpu.sync_copy(acc_v, shared.at[sid])

        plsc.subcore_barrier()

    @pl.when(sid == 0)
    def _():
        pltpu.sync_copy(wp_h, wp_v)
        lane = lax.iota(jnp.int32, L)
        lv = zeros
        for g in range(G):
            def _c(c, pv, g=g):
                return pv + acc_v[g, pl.ds(c * L, L)] * wp_v[pl.ds(c * L, L)]

            pv = lax.fori_loop(0, DC, _c, jnp.zeros((L,), jnp.float32))
            s = pv[0]
            for j in range(1, L):
                s = s + pv[j]
            lv = jnp.where(lane == g, lv + s, lv)
        out_v[...] = lv
        pltpu.sync_copy(out_v, out_h.at[cid])


def _tc_body(idx_ref, x_ref, glob_ref, w_ref, b_ref, log_ref):
    step = pl.program_id(0)
    wp = w_ref[0:D, :]
    oh = (jnp.reshape(idx_ref[...], (1, BT)) ==
          lax.broadcasted_iota(jnp.int32, (G, BT), 0)).astype(jnp.float32)
    part = jnp.dot(oh, x_ref[...], preferred_element_type=jnp.float32)

    @pl.when(step == 0)
    def _():
        wg = w_ref[D:2 * D, :]
        log_ref[...] = (jnp.dot(glob_ref[...], wg,
                                preferred_element_type=jnp.float32) +
                        b_ref[0, 0])

    log_ref[...] += jnp.dot(part, wp, preferred_element_type=jnp.float32)


_tc_pool = pl.pallas_call(
    _tc_body,
    grid=(NBT,),
    in_specs=[
        pl.BlockSpec((1, 1, BT), lambda i: (i, 0, 0)),
        pl.BlockSpec((BT, D), lambda i: (i, 0)),
        pl.BlockSpec((G, D), lambda i: (0, 0)),
        pl.BlockSpec((2 * D, 1), lambda i: (0, 0)),
        pl.BlockSpec((1, 1), lambda i: (0, 0)),
    ],
    out_specs=pl.BlockSpec((G, 1), lambda i: (0, 0)),
    out_shape=jax.ShapeDtypeStruct((G, 1), jnp.float32),
)


def kernel(nodes, edges, senders, receivers, global_latent, node_graph_idx,
           W, b):
    idx = node_graph_idx.astype(jnp.int32)
    wp = W[:D, 0].astype(jnp.float32)
    idx_tc = idx[:NTC].reshape(NBT, 1, BT)
    log_tc = _tc_pool(idx_tc, nodes, global_latent,
                      W.astype(jnp.float32), b.reshape(1, 1))
    parts = _sc_pool(nodes, idx, wp)
    return log_tc + (parts[0] + parts[1]).reshape(G, 1)


# speedup vs baseline: 3.3933x; 1.0941x over previous
import functools

import jax
import jax.numpy as jnp
from jax import lax
from jax.experimental import pallas as pl
from jax.experimental.pallas import tpu as pltpu
from jax.experimental.pallas import tpu_sc as plsc

NC = 2
NS = 16
L = 16
NW = NC * NS
D = 256
G = 16
DC = D // L
N = 50000

BT = 2048
NBT = 17
NTC = NBT * BT
CHUNK = 472
SZS = (120, 120, 120, 112)
BR = 120
TAIL = N - NTC - NW * CHUNK

_mesh = plsc.VectorSubcoreMesh(core_axis_name="c", subcore_axis_name="s")


@functools.partial(
    pl.kernel,
    out_type=jax.ShapeDtypeStruct((NW, G), jnp.float32),
    mesh=_mesh,
    scratch_types=[
        pltpu.VMEM((CHUNK + L,), jnp.int32),
        pltpu.VMEM((TAIL + L,), jnp.int32),
        pltpu.VMEM((3, BR, D), jnp.float32),
        pltpu.VMEM((G, D), jnp.float32),
        pltpu.VMEM((D,), jnp.float32),
        pltpu.VMEM((G,), jnp.float32),
        pltpu.SemaphoreType.DMA,
        pltpu.SemaphoreType.DMA,
        pltpu.SemaphoreType.DMA,
    ],
)
def _sc_pool(nodes_h, idx_h, wp_h, out_h,
             idx_v, idx_t, buf_v, acc_v, wp_v, out_v,
             sem0, sem1, sem2):
    cid = lax.axis_index("c")
    sid = lax.axis_index("s")
    wid = cid * NS + sid
    base = NTC + wid * CHUNK
    sems = (sem0, sem1, sem2)
    offs = [sum(SZS[:k]) for k in range(len(SZS))]

    def _node_copy(blk, b):
        return pltpu.make_async_copy(
            nodes_h.at[pl.ds(base + offs[blk], SZS[blk])],
            buf_v.at[b, pl.ds(0, SZS[blk])],
            sems[b])

    _node_copy(0, 0).start()
    _node_copy(1, 1).start()
    pltpu.sync_copy(idx_h.at[pl.ds(base, CHUNK)], idx_v.at[pl.ds(0, CHUNK)])
    pltpu.sync_copy(wp_h, wp_v)

    zeros = jnp.zeros((L,), jnp.float32)

    def _zero(i, _):
        for c in range(DC):
            acc_v[i, pl.ds(c * L, L)] = zeros
        return 0

    lax.fori_loop(0, G, _zero, 0)

    zregs = tuple(jnp.zeros((L,), jnp.float32) for _ in range(DC))

    def _accum_rows(idx_ref, idx_off, n_rows, b):
        g_first = idx_ref[pl.ds(idx_off, L)][0]
        g_last = idx_ref[pl.ds(idx_off + n_rows - 1, L)][0]

        @pl.when(g_first == g_last)
        def _():
            def _row(i, regs):
                return tuple(regs[c] + buf_v[b, i, pl.ds(c * L, L)]
                             for c in range(DC))

            regs = lax.fori_loop(0, n_rows, _row, zregs, unroll=4)
            for c in range(DC):
                plsc.addupdate(acc_v.at[g_first, pl.ds(c * L, L)], regs[c])

        @pl.when(g_first != g_last)
        def _():
            def _row(i, _):
                g = idx_ref[pl.ds(idx_off + i, L)][0]
                for c in range(DC):
                    plsc.addupdate(acc_v.at[g, pl.ds(c * L, L)],
                                   buf_v[b, i, pl.ds(c * L, L)])
                return 0

            lax.fori_loop(0, n_rows, _row, 0)

    for blk in range(len(SZS)):
        b = blk % 3
        cp = _node_copy(blk, b)
        if blk + 2 < len(SZS):
            _node_copy(blk + 2, (blk + 2) % 3).start()
        cp.wait()
        _accum_rows(idx_v, offs[blk], SZS[blk], b)

    @pl.when(wid == NW - 1)
    def _():
        pltpu.sync_copy(idx_h.at[pl.ds(N - TAIL, TAIL)],
                        idx_t.at[pl.ds(0, TAIL)])
        pltpu.sync_copy(nodes_h.at[pl.ds(N - TAIL, TAIL)],
                        buf_v.at[0, pl.ds(0, TAIL)])
        _accum_rows(idx_t, 0, TAIL, 0)

    lane = lax.iota(jnp.int32, L)
    lv = zeros
    for g in range(G):
        def _c(c, pv, g=g):
            return pv + acc_v[g, pl.ds(c * L, L)] * wp_v[pl.ds(c * L, L)]

        pv = lax.fori_loop(0, DC, _c, jnp.zeros((L,), jnp.float32))
        s = pv[0]
        for j in range(1, L):
            s = s + pv[j]
        lv = jnp.where(lane == g, lv + s, lv)
    out_v[...] = lv
    pltpu.sync_copy(out_v, out_h.at[wid])


def _tc_body(idx_ref, x_ref, glob_ref, w_ref, b_ref, log_ref):
    step = pl.program_id(0)
    wp = w_ref[0:D, :]
    oh = (jnp.reshape(idx_ref[...], (1, BT)) ==
          lax.broadcasted_iota(jnp.int32, (G, BT), 0)).astype(jnp.float32)
    part = jnp.dot(oh, x_ref[...], preferred_element_type=jnp.float32)

    @pl.when(step == 0)
    def _():
        wg = w_ref[D:2 * D, :]
        log_ref[...] = (jnp.dot(glob_ref[...], wg,
                                preferred_element_type=jnp.float32) +
                        b_ref[0, 0])

    log_ref[...] += jnp.dot(part, wp, preferred_element_type=jnp.float32)


_tc_pool = pl.pallas_call(
    _tc_body,
    grid=(NBT,),
    in_specs=[
        pl.BlockSpec((1, 1, BT), lambda i: (i, 0, 0)),
        pl.BlockSpec((BT, D), lambda i: (i, 0)),
        pl.BlockSpec((G, D), lambda i: (0, 0)),
        pl.BlockSpec((2 * D, 1), lambda i: (0, 0)),
        pl.BlockSpec((1, 1), lambda i: (0, 0)),
    ],
    out_specs=pl.BlockSpec((G, 1), lambda i: (0, 0)),
    out_shape=jax.ShapeDtypeStruct((G, 1), jnp.float32),
)


def kernel(nodes, edges, senders, receivers, global_latent, node_graph_idx,
           W, b):
    idx = node_graph_idx.astype(jnp.int32)
    wp = W[:D, 0].astype(jnp.float32)
    idx_tc = idx[:NTC].reshape(NBT, 1, BT)
    log_tc = _tc_pool(idx_tc, nodes, global_latent,
                      W.astype(jnp.float32), b.reshape(1, 1))
    parts = _sc_pool(nodes, idx, wp)
    return log_tc + parts.sum(axis=0).reshape(G, 1)
